# Initial kernel scaffold; baseline (speedup 1.0000x reference)
import jax
import jax.numpy as jnp
from jax.experimental import pallas as pl


def kernel(numeric, f0, f1, f2, f3, f4, f5, f6, f7, f8, f9, f10, f11, f12, f13, f14, f15, f16, f17, f18, f19, f20, f21, f22, f23, f24, f25, emb_0, emb_1, emb_2, emb_3, emb_4, emb_5, emb_6, emb_7, emb_8, emb_9, emb_10, emb_11, emb_12, emb_13, emb_14, emb_15, emb_16, emb_17, emb_18, emb_19, emb_20, emb_21, emb_22, emb_23, emb_24, emb_25, W_wide, b_wide, W1, b1, W2, b2, W_ctr, b_ctr, W_cvr, b_cvr):
    raise NotImplementedError("write your pallas kernel here")


# speedup vs baseline: 2.2272x; 2.2272x over previous
import jax
import jax.numpy as jnp
from jax import lax
from jax.experimental import pallas as pl
from jax.experimental.pallas import tpu as pltpu
from jax.experimental.pallas import tpu_sc as plsc

B = 16384
NUM_DIM = 13
N_FIELDS = 26
EMB_DIM = 16
EMB_COLS = N_FIELDS * EMB_DIM

NC, NS = 2, 16
NW = NC * NS
ROWS_PER_W = B // NW
CHUNK = 128
N_CHUNKS = ROWS_PER_W // CHUNK


def _sc_gather_body(idx_hbm, *rest):
    tables = rest[:N_FIELDS]
    out_hbm = rest[N_FIELDS]
    idx_v, rows_v, gsem, wsem = rest[N_FIELDS + 1:]

    wid = lax.axis_index("s") * NC + lax.axis_index("c")
    base = wid * ROWS_PER_W

    def body(j, carry):
        rbase = base + j * CHUNK
        pltpu.sync_copy(idx_hbm.at[:, pl.ds(rbase, CHUNK)], idx_v)
        gathers = [
            pltpu.async_copy(tables[i].at[idx_v.at[i]], rows_v.at[i], gsem)
            for i in range(N_FIELDS)
        ]
        for c in gathers:
            c.wait()
        writes = [
            pltpu.async_copy(
                rows_v.at[i],
                out_hbm.at[pl.ds(rbase, CHUNK), pl.ds(i * EMB_DIM, EMB_DIM)],
                wsem,
            )
            for i in range(N_FIELDS)
        ]
        for c in writes:
            c.wait()
        return carry

    lax.fori_loop(0, N_CHUNKS, body, 0)


_sc_gather = pl.kernel(
    _sc_gather_body,
    out_type=jax.ShapeDtypeStruct((B, EMB_COLS), jnp.float32),
    mesh=plsc.VectorSubcoreMesh(core_axis_name="c", subcore_axis_name="s"),
    compiler_params=pltpu.CompilerParams(use_tc_tiling_on_sc=False),
    scratch_types=[
        pltpu.VMEM((N_FIELDS, CHUNK), jnp.int32),
        pltpu.VMEM((N_FIELDS, CHUNK, EMB_DIM), jnp.float32),
        pltpu.SemaphoreType.DMA,
        pltpu.SemaphoreType.DMA,
    ],
)

BT = 1024


def _mlp_body(nb, eb, w1n, w1e, b1, w2, b2, ww, bw, wcc, bcc, out):
    x = nb[...]
    h = jnp.maximum(
        jnp.dot(x, w1n[...], preferred_element_type=jnp.float32)
        + jnp.dot(eb[...], w1e[...], preferred_element_type=jnp.float32)
        + b1[...],
        0.0,
    )
    h = jnp.maximum(
        jnp.dot(h, w2[...], preferred_element_type=jnp.float32) + b2[...], 0.0
    )
    wide = jnp.dot(x, ww[...], preferred_element_type=jnp.float32) + bw[...]
    out[...] = (
        jnp.dot(wide, wcc[:16, :], preferred_element_type=jnp.float32)
        + jnp.dot(h, wcc[16:, :], preferred_element_type=jnp.float32)
        + bcc[...]
    )


def _full(shape):
    return pl.BlockSpec(shape, lambda i: (0, 0))


def _mlp(numeric, embedded, w1n, w1e, b1, w2, b2, ww, bw, wcc, bcc):
    return pl.pallas_call(
        _mlp_body,
        grid=(B // BT,),
        in_specs=[
            pl.BlockSpec((BT, NUM_DIM), lambda i: (i, 0)),
            pl.BlockSpec((BT, EMB_COLS), lambda i: (i, 0)),
            _full(w1n.shape),
            _full(w1e.shape),
            _full(b1.shape),
            _full(w2.shape),
            _full(b2.shape),
            _full(ww.shape),
            _full(bw.shape),
            _full(wcc.shape),
            _full(bcc.shape),
        ],
        out_specs=pl.BlockSpec((BT, 2), lambda i: (i, 0)),
        out_shape=jax.ShapeDtypeStruct((B, 2), jnp.float32),
    )(numeric, embedded, w1n, w1e, b1, w2, b2, ww, bw, wcc, bcc)


def kernel(numeric, f0, f1, f2, f3, f4, f5, f6, f7, f8, f9, f10, f11, f12,
           f13, f14, f15, f16, f17, f18, f19, f20, f21, f22, f23, f24, f25,
           emb_0, emb_1, emb_2, emb_3, emb_4, emb_5, emb_6, emb_7, emb_8,
           emb_9, emb_10, emb_11, emb_12, emb_13, emb_14, emb_15, emb_16,
           emb_17, emb_18, emb_19, emb_20, emb_21, emb_22, emb_23, emb_24,
           emb_25,
           W_wide, b_wide, W1, b1, W2, b2, W_ctr, b_ctr, W_cvr, b_cvr):
    fields = [f0, f1, f2, f3, f4, f5, f6, f7, f8, f9, f10, f11, f12, f13,
              f14, f15, f16, f17, f18, f19, f20, f21, f22, f23, f24, f25]
    tables = [emb_0, emb_1, emb_2, emb_3, emb_4, emb_5, emb_6, emb_7, emb_8,
              emb_9, emb_10, emb_11, emb_12, emb_13, emb_14, emb_15, emb_16,
              emb_17, emb_18, emb_19, emb_20, emb_21, emb_22, emb_23, emb_24,
              emb_25]
    idx_all = jnp.stack(fields).astype(jnp.int32)

    embedded = _sc_gather(idx_all, *tables)

    wcc = jnp.concatenate([W_ctr, W_cvr], axis=1)
    bcc = jnp.concatenate([b_ctr, b_cvr]).reshape(1, 2)
    out2 = _mlp(
        numeric, embedded,
        W1[:NUM_DIM], W1[NUM_DIM:], b1.reshape(1, -1),
        W2, b2.reshape(1, -1),
        W_wide, b_wide.reshape(1, -1),
        wcc, bcc,
    )
    return out2[:, 0], out2[:, 1]


# speedup vs baseline: 2.9305x; 1.3158x over previous
import jax
import jax.numpy as jnp
from jax import lax
from jax.experimental import pallas as pl
from jax.experimental.pallas import tpu as pltpu
from jax.experimental.pallas import tpu_sc as plsc

B = 16384
NUM_DIM = 13
N_FIELDS = 26
EMB_DIM = 16
EMB_COLS = N_FIELDS * EMB_DIM
PAD_COLS = 512
VOCAB = 100001

RB = 2048
NSTEPS = (VOCAB + RB - 1) // RB
NR = NSTEPS * RB

NC, NS = 2, 16
NW = NC * NS
ROWS_PER_W = B // NW
CHUNK = 128
N_CHUNKS = ROWS_PER_W // CHUNK


PACK_R = RB // 8
PACK_NR = NR // 8


def _tr_body(*refs):
    ins = refs[:N_FIELDS]
    outs = refs[N_FIELDS:]
    eye = jnp.eye(EMB_DIM, dtype=jnp.float32)
    place = [
        jnp.eye(EMB_DIM, 128, k=16 * s, dtype=jnp.float32) for s in range(8)
    ]
    for t in range(N_FIELDS):
        z = lax.dot_general(
            ins[t][...], eye, (((0,), (0,)), ((), ())),
            preferred_element_type=jnp.float32)
        z3 = z.reshape(PACK_R, 8, EMB_DIM)
        acc = jnp.zeros((PACK_R, 128), jnp.float32)
        for s in range(8):
            zs = lax.squeeze(
                lax.slice(z3, (0, s, 0), (PACK_R, s + 1, EMB_DIM)), (1,))
            acc = acc + jnp.dot(zs, place[s],
                                preferred_element_type=jnp.float32)
        outs[t][...] = acc


def _transpose_pack(tts):
    return pl.pallas_call(
        _tr_body,
        grid=(NSTEPS,),
        in_specs=[pl.BlockSpec((EMB_DIM, RB), lambda i: (0, i))
                  for _ in range(N_FIELDS)],
        out_specs=[pl.BlockSpec((PACK_R, 128), lambda i: (i, 0))
                   for _ in range(N_FIELDS)],
        out_shape=[jax.ShapeDtypeStruct((PACK_NR, 128), jnp.float32)
                   for _ in range(N_FIELDS)],
    )(*tts)


def _sc_gather_body(idx_hbm, *rest):
    tables = rest[:N_FIELDS]
    out_hbm = rest[N_FIELDS]
    idx_v, rows_v, gsem, wsem = rest[N_FIELDS + 1:]

    wid = lax.axis_index("s") * NC + lax.axis_index("c")
    base = wid * ROWS_PER_W

    def body(j, carry):
        rbase = base + j * CHUNK
        pltpu.sync_copy(idx_hbm.at[:, pl.ds(rbase, CHUNK)], idx_v)
        gathers = [
            pltpu.async_copy(tables[i].at[idx_v.at[i]], rows_v.at[i], gsem)
            for i in range(N_FIELDS)
        ]
        for c in gathers:
            c.wait()
        writes = [
            pltpu.async_copy(
                rows_v.at[i],
                out_hbm.at[pl.ds(rbase, CHUNK), pl.ds(i * EMB_DIM, EMB_DIM)],
                wsem,
            )
            for i in range(N_FIELDS)
        ]
        for c in writes:
            c.wait()
        return carry

    lax.fori_loop(0, N_CHUNKS, body, 0)


_sc_gather = pl.kernel(
    _sc_gather_body,
    out_type=jax.ShapeDtypeStruct((B, PAD_COLS), jnp.float32),
    mesh=plsc.VectorSubcoreMesh(core_axis_name="c", subcore_axis_name="s"),
    compiler_params=pltpu.CompilerParams(use_tc_tiling_on_sc=False),
    scratch_types=[
        pltpu.VMEM((N_FIELDS, CHUNK), jnp.int32),
        pltpu.VMEM((N_FIELDS, CHUNK, EMB_DIM), jnp.float32),
        pltpu.SemaphoreType.DMA,
        pltpu.SemaphoreType.DMA,
    ],
)

BT = 1024


def _mlp_body(nb, eb, w1n, w1e, b1, w2, b2, ww, bw, wcc, bcc, out):
    x = nb[...]
    lane = lax.broadcasted_iota(jnp.int32, (1, PAD_COLS), 1)
    e = jnp.where(lane < EMB_COLS, eb[...], 0.0)
    h = jnp.maximum(
        jnp.dot(x, w1n[...], preferred_element_type=jnp.float32)
        + jnp.dot(e, w1e[...], preferred_element_type=jnp.float32)
        + b1[...],
        0.0,
    )
    h = jnp.maximum(
        jnp.dot(h, w2[...], preferred_element_type=jnp.float32) + b2[...], 0.0
    )
    wide = jnp.dot(x, ww[...], preferred_element_type=jnp.float32) + bw[...]
    out[...] = (
        jnp.dot(wide, wcc[:16, :], preferred_element_type=jnp.float32)
        + jnp.dot(h, wcc[16:, :], preferred_element_type=jnp.float32)
        + bcc[...]
    )


def _full(shape):
    return pl.BlockSpec(shape, lambda i: (0, 0))


def _mlp(numeric, embedded, w1n, w1e, b1, w2, b2, ww, bw, wcc, bcc):
    return pl.pallas_call(
        _mlp_body,
        grid=(B // BT,),
        in_specs=[
            pl.BlockSpec((BT, NUM_DIM), lambda i: (i, 0)),
            pl.BlockSpec((BT, PAD_COLS), lambda i: (i, 0)),
            _full(w1n.shape),
            _full(w1e.shape),
            _full(b1.shape),
            _full(w2.shape),
            _full(b2.shape),
            _full(ww.shape),
            _full(bw.shape),
            _full(wcc.shape),
            _full(bcc.shape),
        ],
        out_specs=pl.BlockSpec((BT, 2), lambda i: (i, 0)),
        out_shape=jax.ShapeDtypeStruct((B, 2), jnp.float32),
    )(numeric, embedded, w1n, w1e, b1, w2, b2, ww, bw, wcc, bcc)


def kernel(numeric, f0, f1, f2, f3, f4, f5, f6, f7, f8, f9, f10, f11, f12,
           f13, f14, f15, f16, f17, f18, f19, f20, f21, f22, f23, f24, f25,
           emb_0, emb_1, emb_2, emb_3, emb_4, emb_5, emb_6, emb_7, emb_8,
           emb_9, emb_10, emb_11, emb_12, emb_13, emb_14, emb_15, emb_16,
           emb_17, emb_18, emb_19, emb_20, emb_21, emb_22, emb_23, emb_24,
           emb_25,
           W_wide, b_wide, W1, b1, W2, b2, W_ctr, b_ctr, W_cvr, b_cvr):
    fields = [f0, f1, f2, f3, f4, f5, f6, f7, f8, f9, f10, f11, f12, f13,
              f14, f15, f16, f17, f18, f19, f20, f21, f22, f23, f24, f25]
    tables = [emb_0, emb_1, emb_2, emb_3, emb_4, emb_5, emb_6, emb_7, emb_8,
              emb_9, emb_10, emb_11, emb_12, emb_13, emb_14, emb_15, emb_16,
              emb_17, emb_18, emb_19, emb_20, emb_21, emb_22, emb_23, emb_24,
              emb_25]
    idx_all = jnp.stack(fields).astype(jnp.int32)

    bigs = _transpose_pack([t.T for t in tables])
    tabs = [b.reshape(NR, EMB_DIM) for b in bigs]
    embedded = _sc_gather(idx_all, *tabs)

    w1e = jnp.zeros((PAD_COLS, 128), jnp.float32).at[:EMB_COLS].set(
        W1[NUM_DIM:])
    wcc = jnp.concatenate([W_ctr, W_cvr], axis=1)
    bcc = jnp.concatenate([b_ctr, b_cvr]).reshape(1, 2)
    out2 = _mlp(
        numeric, embedded,
        W1[:NUM_DIM], w1e, b1.reshape(1, -1),
        W2, b2.reshape(1, -1),
        W_wide, b_wide.reshape(1, -1),
        wcc, bcc,
    )
    return out2[:, 0], out2[:, 1]
